# Initial kernel scaffold; baseline (speedup 1.0000x reference)
import jax
import jax.numpy as jnp
from jax.experimental import pallas as pl


def kernel(speed, quats, times_dif, dir, mes, weight, bias, bias_shift, time_shift, types):
    raise NotImplementedError("write your pallas kernel here")


# speedup vs baseline: 2.2965x; 2.2965x over previous
import numpy as np

import jax
import jax.numpy as jnp
from jax.experimental import pallas as pl

_MIN32 = -2147483648


def _block_body(s0, s1, s2, t0, t1, t2, d2, mes, w, b0, b1, ts, bsh, oh, p, s,
                out):
    f32 = jnp.float32
    ts_v = ts[:, :]
    tssum = ts_v[:, 0:1] + ts_v[:, 1:2] + ts_v[:, 2:3]

    sc0 = s0[:, :] * 1000.0 / t0[:, :]
    sc1 = s1[:, :] * 1000.0 / t1[:, :]
    sc2 = s2[:, :] * 1000.0 / t2[:, :]
    sm = (sc2 * ts_v[:, 0:1] + sc1 * ts_v[:, 1:2] + sc0 * ts_v[:, 2:3]) / tssum
    sm = sm + bsh[:, :] * 0.01

    dn = (((1,), (0,)), ((), ()))
    tile = jax.lax.dot_general(sm, p[:, :], dn, preferred_element_type=f32)
    prod = d2[:, :] * tile
    dotp = jax.lax.dot_general(prod, s[:, :], dn, preferred_element_type=f32)
    bterm = jax.lax.dot_general(b0[:, :], oh[:, :], dn,
                                preferred_element_type=f32)

    mes_est = dotp - mes[:, :] + bterm
    wv = w[:, :]
    masked = mes_est + (wv == 0.0).astype(f32) * 10000000.0
    ind = jnp.sum((wv > 0.0).astype(jnp.int32), axis=1, keepdims=True)
    k = ind // 2

    bits = jax.lax.bitcast_convert_type(masked, jnp.int32)
    mn = jnp.int32(_MIN32)
    g = jnp.where(bits >= 0, bits, mn - bits)

    prefix = jnp.zeros(k.shape, jnp.int32)
    for j in range(31, -1, -1):
        bv = (1 << j) - (1 << 32 if j == 31 else 0)
        cand = prefix | jnp.int32(bv)
        thr = cand ^ mn
        cnt = jnp.sum((g < thr).astype(jnp.int32), axis=1, keepdims=True)
        prefix = jnp.where(cnt <= k, cand, prefix)
    gk = prefix ^ mn
    medbits = jnp.where(gk >= 0, gk, mn - gk)
    med = jax.lax.bitcast_convert_type(medbits, f32)
    med = med * (ind > 0).astype(f32)

    n_lanes = wv.shape[1]
    loss = jnp.sum(jnp.abs(mes_est - med) * wv, axis=1, keepdims=True)
    loss = loss * (1.0 / n_lanes)
    bl = jnp.sum(jnp.abs(b1[:, :] - b0[:, :]), axis=1, keepdims=True)
    out[:, :] = loss + bl


def kernel(speed, quats, times_dif, dir, mes, weight, bias, bias_shift,
           time_shift, types):
    del quats
    tp, n = mes.shape
    blk = 256
    nblk = pl.cdiv(tp, blk)
    f32 = jnp.float32

    sp_ext = jnp.concatenate([speed, speed[-1:]], axis=0)
    td_ext = jnp.concatenate([times_dif, times_dif[-1:]], axis=0)
    s0, s1, s2 = sp_ext[:-2], sp_ext[1:-1], sp_ext[2:]
    t0, t1, t2 = td_ext[:-2], td_ext[1:-1], td_ext[2:]

    d2 = dir.reshape(tp, n * 3)
    b_t = jnp.transpose(bias)
    b_t1 = jnp.concatenate([b_t[1:], b_t[-1:]], axis=0)

    nt = bias.shape[0]
    oh = (jnp.arange(nt, dtype=types.dtype)[:, None]
          == types[None, :]).astype(f32)
    lane = jnp.arange(n * 3)
    p_mat = (lane[None, :] % 3 == jnp.arange(3)[:, None]).astype(f32)
    s_mat = (lane[:, None] // 3 == jnp.arange(n)[None, :]).astype(f32)
    ts2d = time_shift.reshape(1, 3)

    row = lambda w: pl.BlockSpec((blk, w), lambda i: (i, 0))
    rep = lambda a, b: pl.BlockSpec((a, b), lambda i: (0, 0))

    out = pl.pallas_call(
        _block_body,
        grid=(nblk,),
        in_specs=[
            row(3), row(3), row(3), row(1), row(1), row(1),
            row(n * 3), row(n), row(n), row(nt), row(nt),
            rep(1, 3), rep(1, 3), rep(nt, n), rep(3, n * 3), rep(n * 3, n),
        ],
        out_specs=pl.BlockSpec((blk, 1), lambda i: (i, 0)),
        out_shape=jax.ShapeDtypeStruct((nblk * blk, 1), f32),
    )(s0, s1, s2, t0, t1, t2, d2, mes, weight, b_t, b_t1,
      ts2d, bias_shift, oh, p_mat, s_mat)

    loss = out[:tp, 0]
    return jnp.concatenate([jnp.zeros((1,), f32), loss], axis=0)


# speedup vs baseline: 3.2502x; 1.4153x over previous
import numpy as np

import jax
import jax.numpy as jnp
from jax.experimental import pallas as pl

_MIN32 = -2147483648


def _block_body(s0, s1, s2, t0, t1, t2, d2, mes_t, w_t, bias0, bias1, ts, bsh,
                oh, p, s, out):
    f32 = jnp.float32
    ts_v = ts[:, :]
    tssum = ts_v[:, 0:1] + ts_v[:, 1:2] + ts_v[:, 2:3]

    sc0 = s0[:, :] * 1000.0 / t0[:, :]
    sc1 = s1[:, :] * 1000.0 / t1[:, :]
    sc2 = s2[:, :] * 1000.0 / t2[:, :]
    sm = (sc2 * ts_v[:, 0:1] + sc1 * ts_v[:, 1:2] + sc0 * ts_v[:, 2:3]) / tssum
    sm = sm + bsh[:, :] * 0.01

    dn = (((1,), (0,)), ((), ()))
    dn00 = (((0,), (0,)), ((), ()))
    tile = jax.lax.dot_general(sm, p[:, :], dn, preferred_element_type=f32)
    prod = d2[:, :] * tile
    dotp_t = jax.lax.dot_general(s[:, :], prod, (((0,), (1,)), ((), ())),
                                 preferred_element_type=f32)
    bterm_t = jax.lax.dot_general(oh[:, :], bias0[:, :], dn00,
                                  preferred_element_type=f32)

    mes_est = dotp_t - mes_t[:, :] + bterm_t
    wv = w_t[:, :]
    masked = mes_est + (wv == 0.0).astype(f32) * 10000000.0
    ind = jnp.sum((wv > 0.0).astype(jnp.int32), axis=0, keepdims=True)
    k = ind // 2

    bits = jax.lax.bitcast_convert_type(masked, jnp.int32)
    mn = jnp.int32(_MIN32)
    g = jnp.where(bits >= 0, bits, mn - bits)

    cnt31 = jnp.sum((g < 0).astype(jnp.int32), axis=0, keepdims=True)
    q = jnp.where(cnt31 <= k, jnp.int32(0), mn)
    for j in range(30, -1, -1):
        thr = q | jnp.int32(1 << j)
        cnt = jnp.sum((g < thr).astype(jnp.int32), axis=0, keepdims=True)
        q = jnp.where(cnt <= k, thr, q)

    medbits = jnp.where(q >= 0, q, mn - q)
    med = jax.lax.bitcast_convert_type(medbits, f32)
    med = med * (ind > 0).astype(f32)

    n_rows = wv.shape[0]
    loss = jnp.sum(jnp.abs(masked - med) * wv, axis=0, keepdims=True)
    loss = loss * (1.0 / n_rows)
    bl = jnp.sum(jnp.abs(bias1[:, :] - bias0[:, :]), axis=0, keepdims=True)
    out[0, :, :] = loss + bl


def kernel(speed, quats, times_dif, dir, mes, weight, bias, bias_shift,
           time_shift, types):
    del quats
    tp, n = mes.shape
    blk = 256
    nblk = pl.cdiv(tp, blk)
    f32 = jnp.float32

    sp_ext = jnp.concatenate([speed, speed[-1:]], axis=0)
    td_ext = jnp.concatenate([times_dif, times_dif[-1:]], axis=0)
    s0, s1, s2 = sp_ext[:-2], sp_ext[1:-1], sp_ext[2:]
    t0, t1, t2 = td_ext[:-2], td_ext[1:-1], td_ext[2:]

    d2 = dir.reshape(tp, n * 3)
    mes_t = jnp.transpose(mes)
    w_t = jnp.transpose(weight)
    bias1 = jnp.concatenate([bias[:, 1:], bias[:, -1:]], axis=1)

    nt = bias.shape[0]
    oh = (jnp.arange(nt, dtype=types.dtype)[:, None]
          == types[None, :]).astype(f32)
    lane = jnp.arange(n * 3)
    p_mat = (lane[None, :] % 3 == jnp.arange(3)[:, None]).astype(f32)
    s_mat = (lane[:, None] // 3 == jnp.arange(n)[None, :]).astype(f32)
    ts2d = time_shift.reshape(1, 3)

    row = lambda w: pl.BlockSpec((blk, w), lambda i: (i, 0))
    col = lambda h: pl.BlockSpec((h, blk), lambda i: (0, i))
    rep = lambda a, b: pl.BlockSpec((a, b), lambda i: (0, 0))

    out = pl.pallas_call(
        _block_body,
        grid=(nblk,),
        in_specs=[
            row(3), row(3), row(3), row(1), row(1), row(1),
            row(n * 3), col(n), col(n), col(nt), col(nt),
            rep(1, 3), rep(1, 3), rep(nt, n), rep(3, n * 3), rep(n * 3, n),
        ],
        out_specs=pl.BlockSpec((1, 1, blk), lambda i: (i, 0, 0)),
        out_shape=jax.ShapeDtypeStruct((nblk, 1, blk), f32),
    )(s0, s1, s2, t0, t1, t2, d2, mes_t, w_t, bias, bias1,
      ts2d, bias_shift, oh, p_mat, s_mat)

    loss = out.reshape(nblk * blk)[:tp]
    return jnp.concatenate([jnp.zeros((1,), f32), loss], axis=0)
